# baseline (speedup 1.0000x reference)
import functools

import jax
import jax.numpy as jnp
from jax import lax
from jax.experimental import pallas as pl
from jax.experimental.pallas import tpu as pltpu
from jax.experimental.pallas import tpu_sc as plsc

B = 4
S = 4096
D = 768
EPS = 1e-12
NC = 2
NS = 16
NW = NC * NS
POS_PER_W = S // NW
T = 32
NSUB = POS_PER_W // T
NCHUNK = NSUB * B
NVREG = D // 16


def _lane_allreduce_sum(x):
    lanes = lax.iota(jnp.int32, 16)
    for k in (8, 4, 2, 1):
        x = x + x.at[lanes ^ k].get(mode="promise_in_bounds")
    return x


def _compute_chunk(w_v, pos_v, g_v, b_v, o_v):
    def stats(t):
        z = jnp.zeros((16,), jnp.float32)

        @plsc.parallel_loop(0, NVREG // 2, carry=(z, z, z, z))
        def p1(j, acc):
            s0, s1, q0, q1 = acc
            d0 = pl.ds(j * 32, 16)
            d1 = pl.ds(j * 32 + 16, 16)
            x0 = w_v[t, d0] + pos_v[t, d0]
            x1 = w_v[t, d1] + pos_v[t, d1]
            o_v[t, d0] = x0
            o_v[t, d1] = x1
            return (s0 + x0, s1 + x1, q0 + x0 * x0, q1 + x1 * x1)

        s0, s1, q0, q1 = p1
        s = _lane_allreduce_sum(s0 + s1)
        q = _lane_allreduce_sum(q0 + q1)
        inv_d = jnp.float32(1.0 / D)
        mean_v = s * inv_d
        var_v = q * inv_d - mean_v * mean_v + jnp.float32(EPS)
        i = lax.bitcast_convert_type(var_v, jnp.int32)
        y = lax.bitcast_convert_type(jnp.int32(0x5F3759DF) - (i >> 1),
                                     jnp.float32)
        half_var = var_v * jnp.float32(0.5)
        for _ in range(3):
            y = y * (jnp.float32(1.5) - half_var * y * y)
        return mean_v, y

    def pair_body(tp, c):
        t0 = tp * 2
        t1 = t0 + 1
        mean0, y0 = stats(t0)
        mean1, y1 = stats(t1)

        @plsc.parallel_loop(0, NVREG, unroll=2)
        def p2(j):
            ds = pl.ds(j * 16, 16)
            gv = g_v[ds]
            bv = b_v[ds]
            t0j = y0 * gv
            c0j = bv - mean0 * t0j
            t1j = y1 * gv
            c1j = bv - mean1 * t1j
            x0 = o_v[t0, ds]
            x1 = o_v[t1, ds]
            o_v[t0, ds] = x0 * t0j + c0j
            o_v[t1, ds] = x1 * t1j + c1j

        return c

    lax.fori_loop(0, T // 2, pair_body, 0)


def _body(ids_hbm, wt_hbm, pt_hbm, g_hbm, bt_hbm, out_hbm,
          idx_v, g_v, b_v, pos_v, w_a, w_b, o_a, o_b,
          gsem_a, gsem_b, osem_a, osem_b):
    cid = lax.axis_index("c")
    sid = lax.axis_index("s")
    wid = cid * NS + sid
    wpos = wid * POS_PER_W

    pltpu.sync_copy(g_hbm, g_v)
    pltpu.sync_copy(bt_hbm, b_v)
    pltpu.sync_copy(ids_hbm.at[:, pl.ds(wpos, POS_PER_W)], idx_v)

    def idx_slice(c):
        return idx_v.at[c % B, pl.ds((c // B) * T, T)]

    def out_slice(c):
        return out_hbm.at[c % B, pl.ds(wpos + (c // B) * T, T)]

    pltpu.async_copy(wt_hbm.at[idx_slice(0)], w_a, gsem_a)

    def chunk(c, w_v, o_v, gsem, osem, gsem_nxt, w_nxt, last):
        @pl.when(c % B == 0)
        def _():
            pltpu.sync_copy(pt_hbm.at[pl.ds(wpos + (c // B) * T, T)], pos_v)

        pltpu.make_async_copy(wt_hbm.at[idx_slice(c)], w_v, gsem).wait()
        if not last:
            pltpu.async_copy(wt_hbm.at[idx_slice(c + 1)], w_nxt, gsem_nxt)
        @pl.when(c >= 2)
        def _():
            pltpu.make_async_copy(o_v, out_slice(c - 2), osem).wait()

        _compute_chunk(w_v, pos_v, g_v, b_v, o_v)
        pltpu.async_copy(o_v, out_slice(c), osem)

    def pair(i, carry):
        cc = i * 2
        chunk(cc, w_a, o_a, gsem_a, osem_a, gsem_b, w_b, False)
        chunk(cc + 1, w_b, o_b, gsem_b, osem_b, gsem_a, w_a,
              False)
        return carry

    lax.fori_loop(0, (NCHUNK - 2) // 2, pair, 0)
    chunk(NCHUNK - 2, w_a, o_a, gsem_a, osem_a, gsem_b, w_b, False)
    chunk(NCHUNK - 1, w_b, o_b, gsem_b, osem_b, gsem_a, w_a, True)
    pltpu.make_async_copy(o_a, out_slice(NCHUNK - 2), osem_a).wait()
    pltpu.make_async_copy(o_b, out_slice(NCHUNK - 1), osem_b).wait()


@jax.jit
def _run(input_ids, word_table, pos_table, gamma, beta):
    mesh = plsc.VectorSubcoreMesh(core_axis_name="c", subcore_axis_name="s")
    kern = functools.partial(
        pl.kernel,
        out_type=jax.ShapeDtypeStruct((B, S, D), jnp.float32),
        mesh=mesh,
        scratch_types=[
            pltpu.VMEM((B, POS_PER_W), jnp.int32),
            pltpu.VMEM((D,), jnp.float32),
            pltpu.VMEM((D,), jnp.float32),
            pltpu.VMEM((T, D), jnp.float32),
            pltpu.VMEM((T, D), jnp.float32),
            pltpu.VMEM((T, D), jnp.float32),
            pltpu.VMEM((T, D), jnp.float32),
            pltpu.VMEM((T, D), jnp.float32),
            pltpu.SemaphoreType.DMA,
            pltpu.SemaphoreType.DMA,
            pltpu.SemaphoreType.DMA,
            pltpu.SemaphoreType.DMA,
        ],
    )(_body)
    return kern(input_ids, word_table, pos_table, gamma, beta)


def kernel(input_ids, word_table, pos_table, gamma, beta):
    ids = input_ids.astype(jnp.int32)
    return _run(ids, word_table, pos_table, gamma, beta)


# speedup vs baseline: 1.2493x; 1.2493x over previous
import functools

import jax
import jax.numpy as jnp
from jax import lax
from jax.experimental import pallas as pl
from jax.experimental.pallas import tpu as pltpu
from jax.experimental.pallas import tpu_sc as plsc

B = 4
S = 4096
D = 768
EPS = 1e-12
NC = 2
NS = 16
NW = NC * NS
POS_PER_W = S // NW
T = 32
NSUB = POS_PER_W // T
NCHUNK = NSUB * B
NVREG = D // 16


def _lane_allreduce_sum(x):
    lanes = lax.iota(jnp.int32, 16)
    for k in (8, 4, 2, 1):
        x = x + x.at[lanes ^ k].get(mode="promise_in_bounds")
    return x


def _finalize(s, q):
    s = _lane_allreduce_sum(s)
    q = _lane_allreduce_sum(q)
    inv_d = jnp.float32(1.0 / D)
    mean_v = s * inv_d
    var_v = q * inv_d - mean_v * mean_v + jnp.float32(EPS)
    i = lax.bitcast_convert_type(var_v, jnp.int32)
    y = lax.bitcast_convert_type(jnp.int32(0x5F3759DF) - (i >> 1),
                                 jnp.float32)
    half_var = var_v * jnp.float32(0.5)
    for _ in range(3):
        y = y * (jnp.float32(1.5) - half_var * y * y)
    return mean_v, y


def _compute_chunk(w_v, pos_v, g_v, b_v, o_v):
    def pair_body(tp, c):
        t0 = tp * 2
        t1 = t0 + 1

        def p1(j, acc):
            a0, a1, b0, b1, c0, c1, d0_, d1_ = acc
            ds0 = pl.ds(j * 32, 16)
            ds1 = pl.ds(j * 32 + 16, 16)
            x00 = w_v[t0, ds0] + pos_v[t0, ds0]
            x01 = w_v[t0, ds1] + pos_v[t0, ds1]
            x10 = w_v[t1, ds0] + pos_v[t1, ds0]
            x11 = w_v[t1, ds1] + pos_v[t1, ds1]
            return (a0 + x00, a1 + x01, b0 + x00 * x00, b1 + x01 * x01,
                    c0 + x10, c1 + x11, d0_ + x10 * x10, d1_ + x11 * x11)

        z = jnp.zeros((16,), jnp.float32)
        a0, a1, b0, b1, c0, c1, d0_, d1_ = lax.fori_loop(
            0, NVREG // 2, p1, (z,) * 8)
        mean0, y0 = _finalize(a0 + a1, b0 + b1)
        mean1, y1 = _finalize(c0 + c1, d0_ + d1_)

        @plsc.parallel_loop(0, NVREG, unroll=2)
        def p2(j):
            ds = pl.ds(j * 16, 16)
            gv = g_v[ds]
            bv = b_v[ds]
            t0j = y0 * gv
            c0j = bv - mean0 * t0j
            t1j = y1 * gv
            c1j = bv - mean1 * t1j
            x0 = w_v[t0, ds] + pos_v[t0, ds]
            x1 = w_v[t1, ds] + pos_v[t1, ds]
            o_v[t0, ds] = x0 * t0j + c0j
            o_v[t1, ds] = x1 * t1j + c1j

        return c

    lax.fori_loop(0, T // 2, pair_body, 0)


def _body(ids_hbm, wt_hbm, pt_hbm, g_hbm, bt_hbm, out_hbm,
          idx_v, g_v, b_v, pos_v, w_a, w_b, o_a, o_b,
          gsem_a, gsem_b, osem_a, osem_b):
    cid = lax.axis_index("c")
    sid = lax.axis_index("s")
    wid = cid * NS + sid
    wpos = wid * POS_PER_W

    pltpu.sync_copy(g_hbm, g_v)
    pltpu.sync_copy(bt_hbm, b_v)
    pltpu.sync_copy(ids_hbm.at[:, pl.ds(wpos, POS_PER_W)], idx_v)

    def idx_slice(c):
        return idx_v.at[c % B, pl.ds((c // B) * T, T)]

    def out_slice(c):
        return out_hbm.at[c % B, pl.ds(wpos + (c // B) * T, T)]

    pltpu.async_copy(wt_hbm.at[idx_slice(0)], w_a, gsem_a)

    def chunk(c, w_v, o_v, gsem, osem, gsem_nxt, w_nxt, last):
        @pl.when(c % B == 0)
        def _():
            pltpu.sync_copy(pt_hbm.at[pl.ds(wpos + (c // B) * T, T)], pos_v)

        pltpu.make_async_copy(wt_hbm.at[idx_slice(c)], w_v, gsem).wait()
        if not last:
            pltpu.async_copy(wt_hbm.at[idx_slice(c + 1)], w_nxt, gsem_nxt)
        @pl.when(c >= 2)
        def _():
            pltpu.make_async_copy(o_v, out_slice(c - 2), osem).wait()

        _compute_chunk(w_v, pos_v, g_v, b_v, o_v)
        pltpu.async_copy(o_v, out_slice(c), osem)

    def pair(i, carry):
        cc = i * 2
        chunk(cc, w_a, o_a, gsem_a, osem_a, gsem_b, w_b, False)
        chunk(cc + 1, w_b, o_b, gsem_b, osem_b, gsem_a, w_a,
              False)
        return carry

    lax.fori_loop(0, (NCHUNK - 2) // 2, pair, 0)
    chunk(NCHUNK - 2, w_a, o_a, gsem_a, osem_a, gsem_b, w_b, False)
    chunk(NCHUNK - 1, w_b, o_b, gsem_b, osem_b, gsem_a, w_a, True)
    pltpu.make_async_copy(o_a, out_slice(NCHUNK - 2), osem_a).wait()
    pltpu.make_async_copy(o_b, out_slice(NCHUNK - 1), osem_b).wait()


@jax.jit
def _run(input_ids, word_table, pos_table, gamma, beta):
    mesh = plsc.VectorSubcoreMesh(core_axis_name="c", subcore_axis_name="s")
    kern = functools.partial(
        pl.kernel,
        out_type=jax.ShapeDtypeStruct((B, S, D), jnp.float32),
        mesh=mesh,
        scratch_types=[
            pltpu.VMEM((B, POS_PER_W), jnp.int32),
            pltpu.VMEM((D,), jnp.float32),
            pltpu.VMEM((D,), jnp.float32),
            pltpu.VMEM((T, D), jnp.float32),
            pltpu.VMEM((T, D), jnp.float32),
            pltpu.VMEM((T, D), jnp.float32),
            pltpu.VMEM((T, D), jnp.float32),
            pltpu.VMEM((T, D), jnp.float32),
            pltpu.SemaphoreType.DMA,
            pltpu.SemaphoreType.DMA,
            pltpu.SemaphoreType.DMA,
            pltpu.SemaphoreType.DMA,
        ],
    )(_body)
    return kern(input_ids, word_table, pos_table, gamma, beta)


def kernel(input_ids, word_table, pos_table, gamma, beta):
    ids = input_ids.astype(jnp.int32)
    return _run(ids, word_table, pos_table, gamma, beta)


# speedup vs baseline: 1.3604x; 1.0890x over previous
import functools

import jax
import jax.numpy as jnp
from jax import lax
from jax.experimental import pallas as pl
from jax.experimental.pallas import tpu as pltpu
from jax.experimental.pallas import tpu_sc as plsc

B = 4
S = 4096
D = 768
EPS = 1e-12
NC = 2
NS = 16
NW = NC * NS
POS_PER_W = S // NW
T = 32
NSUB = POS_PER_W // T
NCHUNK = NSUB * B
NVREG = D // 16


def _lane_allreduce_sum(x):
    lanes = lax.iota(jnp.int32, 16)
    for k in (8, 4, 2, 1):
        x = x + x.at[lanes ^ k].get(mode="promise_in_bounds")
    return x


def _finalize(s, q):
    s = _lane_allreduce_sum(s)
    q = _lane_allreduce_sum(q)
    inv_d = jnp.float32(1.0 / D)
    mean_v = s * inv_d
    var_v = q * inv_d - mean_v * mean_v + jnp.float32(EPS)
    i = lax.bitcast_convert_type(var_v, jnp.int32)
    y = lax.bitcast_convert_type(jnp.int32(0x5F3759DF) - (i >> 1),
                                 jnp.float32)
    half_var = var_v * jnp.float32(0.5)
    for _ in range(3):
        y = y * (jnp.float32(1.5) - half_var * y * y)
    return mean_v, y


def _compute_chunk(w_v, pos_v, g_v, b_v, o_v):
    def quad_body(tq, c):
        ts = [tq * 4 + k for k in range(4)]

        def p1(j, acc):
            ds = pl.ds(j * 16, 16)
            new = []
            for k in range(4):
                x = w_v[ts[k], ds] + pos_v[ts[k], ds]
                new.append(acc[k] + x)
                new.append(acc[4 + k] + x * x)
            return (new[0], new[2], new[4], new[6],
                    new[1], new[3], new[5], new[7])

        z = jnp.zeros((16,), jnp.float32)
        acc = lax.fori_loop(0, NVREG, p1, (z,) * 8)
        my = [_finalize(acc[k], acc[4 + k]) for k in range(4)]

        @plsc.parallel_loop(0, NVREG, unroll=2)
        def p2(j):
            ds = pl.ds(j * 16, 16)
            gv = g_v[ds]
            bv = b_v[ds]
            for k in range(4):
                mk, yk = my[k]
                tk = yk * gv
                ck = bv - mk * tk
                x = w_v[ts[k], ds] + pos_v[ts[k], ds]
                o_v[ts[k], ds] = x * tk + ck

        return c

    lax.fori_loop(0, T // 4, quad_body, 0)


def _body(ids_hbm, wt_hbm, pt_hbm, g_hbm, bt_hbm, out_hbm,
          idx_v, g_v, b_v, pos_v, w_a, w_b, o_a, o_b,
          gsem_a, gsem_b, osem_a, osem_b):
    cid = lax.axis_index("c")
    sid = lax.axis_index("s")
    wid = cid * NS + sid
    wpos = wid * POS_PER_W

    pltpu.sync_copy(g_hbm, g_v)
    pltpu.sync_copy(bt_hbm, b_v)
    pltpu.sync_copy(ids_hbm.at[:, pl.ds(wpos, POS_PER_W)], idx_v)

    def idx_slice(c):
        return idx_v.at[c % B, pl.ds((c // B) * T, T)]

    def out_slice(c):
        return out_hbm.at[c % B, pl.ds(wpos + (c // B) * T, T)]

    pltpu.async_copy(wt_hbm.at[idx_slice(0)], w_a, gsem_a)

    def chunk(c, w_v, o_v, gsem, osem, gsem_nxt, w_nxt, last):
        @pl.when(c % B == 0)
        def _():
            pltpu.sync_copy(pt_hbm.at[pl.ds(wpos + (c // B) * T, T)], pos_v)

        pltpu.make_async_copy(wt_hbm.at[idx_slice(c)], w_v, gsem).wait()
        if not last:
            pltpu.async_copy(wt_hbm.at[idx_slice(c + 1)], w_nxt, gsem_nxt)
        @pl.when(c >= 2)
        def _():
            pltpu.make_async_copy(o_v, out_slice(c - 2), osem).wait()

        _compute_chunk(w_v, pos_v, g_v, b_v, o_v)
        pltpu.async_copy(o_v, out_slice(c), osem)

    def pair(i, carry):
        cc = i * 2
        chunk(cc, w_a, o_a, gsem_a, osem_a, gsem_b, w_b, False)
        chunk(cc + 1, w_b, o_b, gsem_b, osem_b, gsem_a, w_a,
              False)
        return carry

    lax.fori_loop(0, (NCHUNK - 2) // 2, pair, 0)
    chunk(NCHUNK - 2, w_a, o_a, gsem_a, osem_a, gsem_b, w_b, False)
    chunk(NCHUNK - 1, w_b, o_b, gsem_b, osem_b, gsem_a, w_a, True)
    pltpu.make_async_copy(o_a, out_slice(NCHUNK - 2), osem_a).wait()
    pltpu.make_async_copy(o_b, out_slice(NCHUNK - 1), osem_b).wait()


@jax.jit
def _run(input_ids, word_table, pos_table, gamma, beta):
    mesh = plsc.VectorSubcoreMesh(core_axis_name="c", subcore_axis_name="s")
    kern = functools.partial(
        pl.kernel,
        out_type=jax.ShapeDtypeStruct((B, S, D), jnp.float32),
        mesh=mesh,
        scratch_types=[
            pltpu.VMEM((B, POS_PER_W), jnp.int32),
            pltpu.VMEM((D,), jnp.float32),
            pltpu.VMEM((D,), jnp.float32),
            pltpu.VMEM((T, D), jnp.float32),
            pltpu.VMEM((T, D), jnp.float32),
            pltpu.VMEM((T, D), jnp.float32),
            pltpu.VMEM((T, D), jnp.float32),
            pltpu.VMEM((T, D), jnp.float32),
            pltpu.SemaphoreType.DMA,
            pltpu.SemaphoreType.DMA,
            pltpu.SemaphoreType.DMA,
            pltpu.SemaphoreType.DMA,
        ],
    )(_body)
    return kern(input_ids, word_table, pos_table, gamma, beta)


def kernel(input_ids, word_table, pos_table, gamma, beta):
    ids = input_ids.astype(jnp.int32)
    return _run(ids, word_table, pos_table, gamma, beta)


# speedup vs baseline: 1.4427x; 1.0605x over previous
import functools

import jax
import jax.numpy as jnp
from jax import lax
from jax.experimental import pallas as pl
from jax.experimental.pallas import tpu as pltpu
from jax.experimental.pallas import tpu_sc as plsc

B = 4
S = 4096
D = 768
EPS = 1e-12
NC = 2
NS = 16
NW = NC * NS
POS_PER_W = S // NW
T = 32
NSUB = POS_PER_W // T
NCHUNK = NSUB * B
NVREG = D // 16


def _lane_allreduce_sum(x):
    lanes = lax.iota(jnp.int32, 16)
    for k in (8, 4, 2, 1):
        x = x + x.at[lanes ^ k].get(mode="promise_in_bounds")
    return x


def _finalize(s, q):
    s = _lane_allreduce_sum(s)
    q = _lane_allreduce_sum(q)
    inv_d = jnp.float32(1.0 / D)
    mean_v = s * inv_d
    var_v = q * inv_d - mean_v * mean_v + jnp.float32(EPS)
    i = lax.bitcast_convert_type(var_v, jnp.int32)
    y = lax.bitcast_convert_type(jnp.int32(0x5F3759DF) - (i >> 1),
                                 jnp.float32)
    half_var = var_v * jnp.float32(0.5)
    for _ in range(3):
        y = y * (jnp.float32(1.5) - half_var * y * y)
    return mean_v, y


def _compute_chunk(w_v, pos_v, g_v, b_v, o_v):
    NG = 8

    def quad_body(tq, c):
        ts = [tq * NG + k for k in range(NG)]

        def p1(j, acc):
            ds = pl.ds(j * 16, 16)
            sums = list(acc[:NG])
            sqs = list(acc[NG:])
            for k in range(NG):
                x = w_v[ts[k], ds] + pos_v[ts[k], ds]
                sums[k] = sums[k] + x
                sqs[k] = sqs[k] + x * x
            return tuple(sums) + tuple(sqs)

        z = jnp.zeros((16,), jnp.float32)
        acc = lax.fori_loop(0, NVREG, p1, (z,) * (2 * NG))
        my = [_finalize(acc[k], acc[NG + k]) for k in range(NG)]

        @plsc.parallel_loop(0, NVREG, unroll=2)
        def p2(j):
            ds = pl.ds(j * 16, 16)
            gv = g_v[ds]
            bv = b_v[ds]
            for k in range(NG):
                mk, yk = my[k]
                tk = yk * gv
                ck = bv - mk * tk
                x = w_v[ts[k], ds] + pos_v[ts[k], ds]
                o_v[ts[k], ds] = x * tk + ck

        return c

    lax.fori_loop(0, T // NG, quad_body, 0)


def _body(ids_hbm, wt_hbm, pt_hbm, g_hbm, bt_hbm, out_hbm,
          idx_v, g_v, b_v, pos_v, w_a, w_b, o_a, o_b,
          gsem_a, gsem_b, osem_a, osem_b):
    cid = lax.axis_index("c")
    sid = lax.axis_index("s")
    wid = cid * NS + sid
    wpos = wid * POS_PER_W

    pltpu.sync_copy(g_hbm, g_v)
    pltpu.sync_copy(bt_hbm, b_v)
    pltpu.sync_copy(ids_hbm.at[:, pl.ds(wpos, POS_PER_W)], idx_v)

    def idx_slice(c):
        return idx_v.at[c % B, pl.ds((c // B) * T, T)]

    def out_slice(c):
        return out_hbm.at[c % B, pl.ds(wpos + (c // B) * T, T)]

    pltpu.async_copy(wt_hbm.at[idx_slice(0)], w_a, gsem_a)

    def chunk(c, w_v, o_v, gsem, osem, gsem_nxt, w_nxt, last):
        @pl.when(c % B == 0)
        def _():
            pltpu.sync_copy(pt_hbm.at[pl.ds(wpos + (c // B) * T, T)], pos_v)

        pltpu.make_async_copy(wt_hbm.at[idx_slice(c)], w_v, gsem).wait()
        if not last:
            pltpu.async_copy(wt_hbm.at[idx_slice(c + 1)], w_nxt, gsem_nxt)
        @pl.when(c >= 2)
        def _():
            pltpu.make_async_copy(o_v, out_slice(c - 2), osem).wait()

        _compute_chunk(w_v, pos_v, g_v, b_v, o_v)
        pltpu.async_copy(o_v, out_slice(c), osem)

    def pair(i, carry):
        cc = i * 2
        chunk(cc, w_a, o_a, gsem_a, osem_a, gsem_b, w_b, False)
        chunk(cc + 1, w_b, o_b, gsem_b, osem_b, gsem_a, w_a,
              False)
        return carry

    lax.fori_loop(0, (NCHUNK - 2) // 2, pair, 0)
    chunk(NCHUNK - 2, w_a, o_a, gsem_a, osem_a, gsem_b, w_b, False)
    chunk(NCHUNK - 1, w_b, o_b, gsem_b, osem_b, gsem_a, w_a, True)
    pltpu.make_async_copy(o_a, out_slice(NCHUNK - 2), osem_a).wait()
    pltpu.make_async_copy(o_b, out_slice(NCHUNK - 1), osem_b).wait()


@jax.jit
def _run(input_ids, word_table, pos_table, gamma, beta):
    mesh = plsc.VectorSubcoreMesh(core_axis_name="c", subcore_axis_name="s")
    kern = functools.partial(
        pl.kernel,
        out_type=jax.ShapeDtypeStruct((B, S, D), jnp.float32),
        mesh=mesh,
        scratch_types=[
            pltpu.VMEM((B, POS_PER_W), jnp.int32),
            pltpu.VMEM((D,), jnp.float32),
            pltpu.VMEM((D,), jnp.float32),
            pltpu.VMEM((T, D), jnp.float32),
            pltpu.VMEM((T, D), jnp.float32),
            pltpu.VMEM((T, D), jnp.float32),
            pltpu.VMEM((T, D), jnp.float32),
            pltpu.VMEM((T, D), jnp.float32),
            pltpu.SemaphoreType.DMA,
            pltpu.SemaphoreType.DMA,
            pltpu.SemaphoreType.DMA,
            pltpu.SemaphoreType.DMA,
        ],
    )(_body)
    return kern(input_ids, word_table, pos_table, gamma, beta)


def kernel(input_ids, word_table, pos_table, gamma, beta):
    ids = input_ids.astype(jnp.int32)
    return _run(ids, word_table, pos_table, gamma, beta)
